# baseline (speedup 1.0000x reference)
import functools

import jax
import jax.numpy as jnp
from jax import lax
from jax.experimental import pallas as pl
from jax.experimental.pallas import tpu as pltpu
from jax.experimental.pallas import tpu_sc as plsc

EMB = 32
HID = 128
_PACK = HID // EMB

_FIRE = 128
_FPG = 8
_GROUP = _FIRE * _FPG
_CHUNK = _GROUP // _PACK


def _sc_gather(x2d, w):
    n = x2d.shape[0] * x2d.shape[1]
    info = plsc.get_sparse_core_info()
    nc, ns = info.num_cores, info.num_subcores
    nw = nc * ns
    per_w = n // nw
    groups = per_w // _GROUP

    mesh = plsc.VectorSubcoreMesh(core_axis_name="c", subcore_axis_name="s")

    @functools.partial(
        pl.kernel,
        mesh=mesh,
        out_type=jax.ShapeDtypeStruct((n // _PACK, HID), jnp.float32),
        scratch_types=[
            pltpu.VMEM((2, _FPG, _FIRE), jnp.int32),
            pltpu.VMEM((2, _GROUP, EMB), jnp.float32),
            pltpu.SemaphoreType.DMA,
            pltpu.SemaphoreType.DMA,
            pltpu.SemaphoreType.DMA,
        ],
        compiler_params=pltpu.CompilerParams(use_tc_tiling_on_sc=False),
    )
    def gather(x_hbm, w_hbm, out_hbm, idx_v, rows_v, sem_g, sem_i, sem_o):
        wid = lax.axis_index("s") * nc + lax.axis_index("c")
        row_base = wid * (per_w // _FIRE)
        out_base = wid * (per_w // _PACK)

        def fire_group(slot):
            for j in range(_FPG):
                pltpu.async_copy(
                    w_hbm.at[idx_v.at[slot, j]],
                    rows_v.at[slot, pl.ds(j * _FIRE, _FIRE)],
                    sem_g,
                )

        def store_copies(slot, g):
            return [
                (
                    rows_v.at[slot, pl.ds(c * _CHUNK, _CHUNK)],
                    out_hbm.at[
                        pl.ds(out_base + g * _CHUNK, _CHUNK),
                        pl.ds(c * EMB, EMB),
                    ],
                )
                for c in range(_PACK)
            ]

        pltpu.sync_copy(x_hbm.at[pl.ds(row_base, _FPG)], idx_v.at[0])
        fire_group(0)

        def body(g, carry):
            slot = lax.rem(g, 2)
            nslot = 1 - slot

            @pl.when(g + 1 < groups)
            def _():
                pltpu.async_copy(
                    x_hbm.at[pl.ds(row_base + (g + 1) * _FPG, _FPG)],
                    idx_v.at[nslot],
                    sem_i,
                )

            pltpu.make_async_copy(
                out_hbm.at[pl.ds(0, _GROUP), pl.ds(0, EMB)],
                rows_v.at[slot],
                sem_g,
            ).wait()

            @pl.when(g >= 1)
            def _():
                for src, dst in store_copies(nslot, 0):
                    pltpu.make_async_copy(src, dst, sem_o).wait()

            @pl.when(g + 1 < groups)
            def _():
                pltpu.make_async_copy(
                    x_hbm.at[pl.ds(row_base, _FPG)],
                    idx_v.at[nslot],
                    sem_i,
                ).wait()
                fire_group(nslot)

            for src, dst in store_copies(slot, g):
                pltpu.async_copy(src, dst, sem_o)
            return carry

        lax.fori_loop(0, groups, body, 0)

        for src, dst in store_copies((groups - 1) % 2, 0):
            pltpu.make_async_copy(src, dst, sem_o).wait()

    return gather(x2d, w)


def _tc_project(g4, m, b, l):
    bb = 4
    rows = bb * l
    nb = b // bb
    kb = nb // _PACK

    def mm(g4_ref, m_ref, o_ref):
        acc = lax.dot_general(
            g4_ref[...],
            m_ref[0],
            (((1,), (0,)), ((), ())),
            preferred_element_type=jnp.float32,
        )
        o_ref[...] = acc.reshape(bb, l, HID)

    return pl.pallas_call(
        mm,
        grid=(kb, _PACK),
        in_specs=[
            pl.BlockSpec((rows, HID), lambda i, k: (i, 0)),
            pl.BlockSpec((1, HID, HID), lambda i, k: (k, 0, 0)),
        ],
        out_specs=pl.BlockSpec((bb, l, HID), lambda i, k: (k * kb + i, 0, 0)),
        out_shape=jax.ShapeDtypeStruct((b, l, HID), jnp.float32),
    )(g4, m)


def kernel(x, W, We):
    b, l = x.shape
    n = b * l
    xf = x.reshape(-1).astype(jnp.int32)
    xp = (
        xf.reshape(_PACK, (n // _PACK) // _CHUNK, _CHUNK)
        .transpose(1, 0, 2)
        .reshape(n // _FIRE, _FIRE)
    )
    g4 = _sc_gather(xp, W)
    m = jnp.stack(
        [
            jnp.zeros((HID, HID), jnp.float32)
            .at[k * EMB:(k + 1) * EMB, :]
            .set(We.T)
            for k in range(_PACK)
        ]
    )
    return _tc_project(g4, m, b, l)
